# Initial kernel scaffold; baseline (speedup 1.0000x reference)
import jax
import jax.numpy as jnp
from jax.experimental import pallas as pl


def kernel(cw1, cb1, cw2, cb2, cw3, cb3, w1s, w1v, b1, w2, b2, wa, ba, wq, bq, state, vec, action):
    raise NotImplementedError("write your pallas kernel here")


# speedup vs baseline: 5.4979x; 5.4979x over previous
import numpy as np
import jax
import jax.numpy as jnp
from jax.experimental import pallas as pl
from jax.experimental.pallas import tpu as pltpu

_B = 32
_H = _W = 40
_CIN = 4
_KH = _KW = 5

_R1, _C1, _OW1, _P1 = 40, 32, 36, 18
_R2, _C2, _OW2, _P2 = 18, 64, 14, 7
_R3, _C3, _OW3, _P3 = 7, 128, 3, 1

_L1 = _W * _CIN
_L2 = 640
_L3 = 512
_N1 = 2 * _L2
_N2 = 2 * _L3
_N3 = _OW3 * _C3
_K1, _K2, _K3 = _KH * _L1, _KH * _L2, _KH * _L3

_M1, _M2, _M3 = _B * _R1, _B * _R2, _B * _R3
_T1 = _N1 // 128
_T2 = _N2 // 128
_HT1, _HT2 = _T1 // 2, _T2 // 2


def _sel(w_slots, jp_slots, w_real, jp_real):
    s = np.zeros((_KW, w_slots, 2, jp_slots), np.float32)
    for kw in range(_KW):
        for jp in range(jp_real):
            for p in range(2):
                w = 2 * jp + p + kw
                if w < w_real:
                    s[kw, w, p, jp] = 1.0
    return s


_S1 = _sel(_W, 20, _W, _P1)
_S2 = _sel(20, 8, _P1, _P2)
_S3 = np.zeros((_KW, 8, _OW3), np.float32)
for _kw in range(_KW):
    for _ow in range(_OW3):
        if _ow + _kw < _P2:
            _S3[_kw, _ow + _kw, _ow] = 1.0


def _fwd_kernel(x_ref, vec_ref, act_ref,
                w1_ref, b1r_ref, w2_ref, b2r_ref, w3_ref, b3r_ref,
                w1s_ref, w1v_ref, fb1_ref, fw2_ref, fb2_ref,
                wa_ref, ba_ref, wq_ref, bq_ref,
                q_ref, *scr):
    f32 = jnp.float32
    bf16 = jnp.bfloat16
    it = iter(scr)
    xk1 = next(it)
    o1 = [next(it) for _ in range(_T1)]
    a2 = next(it)
    xk2 = next(it)
    o2 = [next(it) for _ in range(_T2)]
    a3 = next(it)
    xk3 = next(it)
    o3 = [next(it) for _ in range(2)]

    for kh in range(_KH):
        xk1[pl.ds(0, _M1 - kh), pl.ds(kh * _L1, _L1)] = x_ref[pl.ds(kh, _M1 - kh), :]
    v = jnp.maximum(
        jnp.dot(xk1[...], w1_ref[...], preferred_element_type=f32)
        + b1r_ref[...], 0.0)
    for t in range(_T1):
        o1[t][...] = v[:, t * 128:(t + 1) * 128]

    for t in range(_HT1):
        p = jnp.maximum(
            jnp.maximum(o1[t][pl.ds(0, _M1 // 2, 2), :],
                        o1[t][pl.ds(1, _M1 // 2, 2), :]),
            jnp.maximum(o1[t + _HT1][pl.ds(0, _M1 // 2, 2), :],
                        o1[t + _HT1][pl.ds(1, _M1 // 2, 2), :])).astype(bf16)
        for b in range(_B):
            a2[pl.ds(b * _R2, _R2), pl.ds(t * 128, 128)] = \
                p[b * _R1 // 2: b * _R1 // 2 + _R2, :]

    for kh in range(_KH):
        xk2[pl.ds(0, _M2 - kh), pl.ds(kh * _L2, _L2)] = a2[pl.ds(kh, _M2 - kh), :]
    v = jnp.maximum(
        jnp.dot(xk2[...], w2_ref[...], preferred_element_type=f32)
        + b2r_ref[...], 0.0)
    for t in range(_T2):
        o2[t][...] = v[:, t * 128:(t + 1) * 128]

    for t in range(_HT2):
        p = jnp.maximum(
            jnp.maximum(o2[t][pl.ds(0, _M2 // 2, 2), :],
                        o2[t][pl.ds(1, _M2 // 2, 2), :]),
            jnp.maximum(o2[t + _HT2][pl.ds(0, _M2 // 2, 2), :],
                        o2[t + _HT2][pl.ds(1, _M2 // 2, 2), :])).astype(bf16)
        for b in range(_B):
            a3[pl.ds(b * _R3, _R3), pl.ds(t * 128, 128)] = \
                p[b * _R2 // 2: b * _R2 // 2 + _R3, :]

    for kh in range(_KH):
        xk3[pl.ds(0, _M3 - kh), pl.ds(kh * _L3, _L3)] = a3[pl.ds(kh, _M3 - kh), :]
    v = jnp.maximum(
        jnp.dot(xk3[...], w3_ref[...], preferred_element_type=f32)
        + b3r_ref[...], 0.0)
    o3[0][...] = v[:, 0:128]
    o3[1][...] = v[:, 128:256]

    flat = jnp.maximum(
        jnp.maximum(o3[0][pl.ds(0, _B, _R3), :], o3[0][pl.ds(1, _B, _R3), :]),
        jnp.maximum(o3[1][pl.ds(0, _B, _R3), :], o3[1][pl.ds(1, _B, _R3), :]))

    h1 = jnp.maximum(
        jnp.dot(flat, w1s_ref[...], preferred_element_type=f32)
        + jnp.dot(vec_ref[...], w1v_ref[...], preferred_element_type=f32)
        + fb1_ref[...], 0.0)
    h2 = (jnp.dot(h1, fw2_ref[...], preferred_element_type=f32) + fb2_ref[...])
    av = jnp.maximum(
        jnp.dot(act_ref[...], wa_ref[...], preferred_element_type=f32)
        + ba_ref[...], 0.0)
    sv = jnp.maximum(h2 + av, 0.0)
    q_ref[...] = (jnp.dot(sv, wq_ref[...], preferred_element_type=f32)
                  + bq_ref[...])


def kernel(cw1, cb1, cw2, cb2, cw3, cb3, w1s, w1v, b1, w2, b2, wa, ba, wq, bq,
           state, vec, action):
    n = state.shape[0]
    f32 = jnp.float32
    bf16 = jnp.bfloat16

    xr = jnp.transpose(state.astype(bf16), (0, 2, 1, 3)).reshape(n * _H, _L1)

    s1, s2, s3 = (jnp.asarray(s).astype(bf16) for s in (_S1, _S2, _S3))
    w51 = cw1.reshape(_KH, _KW, _CIN, _C1).astype(bf16)
    wb1 = jnp.einsum('awpj,kaic->kiwpjc', s1, w51).reshape(_K1, _N1)
    w52 = cw2.reshape(_KH, _KW, _C1, _C2).astype(bf16)
    wb2 = jnp.einsum('awpj,kaic->kwipjc', s2, w52).reshape(_K2, _N2)
    w53 = cw3.reshape(_KH, _KW, _C2, _C3).astype(bf16)
    wb3 = jnp.einsum('awo,kaic->kwioc', s3, w53).reshape(_K3, _N3)

    b1r = jnp.tile(jnp.pad(jnp.tile(cb1, (1, _P1)), ((0, 0), (0, _L2 - _P1 * _C1))), (1, 2))
    b2r = jnp.tile(jnp.pad(jnp.tile(cb2, (1, _P2)), ((0, 0), (0, _L3 - _P2 * _C2))), (1, 2))
    b3r = jnp.tile(cb3, (1, _OW3))

    def rows_spec(r, c):
        return pl.BlockSpec((r, c), lambda i: (i, 0))

    def full_spec(shape):
        return pl.BlockSpec(shape, lambda i: (0,) * len(shape))

    in_specs = [
        rows_spec(_B * _H, _L1),
        rows_spec(_B, 4),
        rows_spec(_B, 2),
        full_spec(wb1.shape), full_spec(b1r.shape),
        full_spec(wb2.shape), full_spec(b2r.shape),
        full_spec(wb3.shape), full_spec(b3r.shape),
        full_spec(w1s.shape), full_spec(w1v.shape), full_spec(b1.shape),
        full_spec(w2.shape), full_spec(b2.shape),
        full_spec(wa.shape), full_spec(ba.shape),
        full_spec(wq.shape), full_spec(bq.shape),
    ]

    scratch = (
        [pltpu.VMEM((_M1, _K1), bf16)]
        + [pltpu.VMEM((_M1, 128), f32) for _ in range(_T1)]
        + [pltpu.VMEM((_M2, _L2), bf16),
           pltpu.VMEM((_M2, _K2), bf16)]
        + [pltpu.VMEM((_M2, 128), f32) for _ in range(_T2)]
        + [pltpu.VMEM((_M3, _L3), bf16),
           pltpu.VMEM((_M3, _K3), bf16)]
        + [pltpu.VMEM((_M3, 128), f32) for _ in range(2)]
    )

    q = pl.pallas_call(
        _fwd_kernel,
        out_shape=jax.ShapeDtypeStruct((n, 1), f32),
        grid=(n // _B,),
        in_specs=in_specs,
        out_specs=rows_spec(_B, 1),
        scratch_shapes=scratch,
        compiler_params=pltpu.CompilerParams(
            dimension_semantics=("parallel",),
            vmem_limit_bytes=100 * 1024 * 1024),
    )(xr, vec.astype(f32), action.astype(f32),
      wb1, b1r, wb2, b2r, wb3, b3r,
      w1s, w1v, b1, w2, b2, wa, ba, wq, bq)
    return q
